# baseline (speedup 1.0000x reference)
import functools

import jax
import jax.numpy as jnp
from jax import lax
from jax.experimental import pallas as pl
from jax.experimental.pallas import tpu as pltpu
from jax.experimental.pallas import tpu_sc as plsc

_STEPS = 3
_NC, _NS = 2, 16
_NW = _NC * _NS
_EB = 128


def _propagate_body(n_nodes, nb, src_hbm, dst_hbm, table_hbm, zeros_hbm,
                    out_hbm, src_v, dst_ring, rows0_v, rows1_v, sem0, sem1,
                    acc_sh):
    c = lax.axis_index("c")
    s = lax.axis_index("s")
    w = s * _NC + c
    rpt = n_nodes // _NS

    pltpu.sync_copy(zeros_hbm, acc_sh.at[pl.ds(s * rpt, rpt)])

    pltpu.sync_copy(src_hbm.at[w], src_v)
    plsc.subcore_barrier()

    rows = (rows0_v, rows1_v)
    sems = (sem0, sem1)

    def start(j, p):
        pltpu.async_copy(dst_hbm.at[w, j], dst_ring.at[p], sems[p])
        pltpu.async_copy(table_hbm.at[src_v.at[j]], rows[p], sems[p])

    def finish(j, p):
        pltpu.make_async_copy(dst_hbm.at[w, j], dst_ring.at[p],
                              sems[p]).wait()
        pltpu.make_async_copy(table_hbm.at[src_v.at[j]], rows[p],
                              sems[p]).wait()
        pltpu.sync_copy(rows[p], acc_sh.at[dst_ring.at[p, 0]], add=True)

    start(0, 0)
    start(1, 1)

    def body(i, _):
        j = 2 * i
        finish(j, 0)

        @pl.when(j + 2 < nb)
        def _():
            start(j + 2, 0)

        finish(j + 1, 1)

        @pl.when(j + 3 < nb)
        def _():
            start(j + 3, 1)

        return 0

    lax.fori_loop(0, nb // 2, body, 0)
    if nb % 2:
        finish(nb - 1, 0)
    plsc.subcore_barrier()

    pltpu.sync_copy(acc_sh.at[pl.ds(s * rpt, rpt)], out_hbm.at[c, s])


def _make_propagate(n_nodes, n_edges_pad, channels):
    nb = n_edges_pad // (_NW * _EB)
    rpt = n_nodes // _NS
    mesh = plsc.VectorSubcoreMesh(core_axis_name="c", subcore_axis_name="s")
    return pl.kernel(
        functools.partial(_propagate_body, n_nodes, nb),
        out_type=jax.ShapeDtypeStruct((_NC, _NS, rpt, channels), jnp.float32),
        mesh=mesh,
        scratch_types=[
            pltpu.VMEM((nb, _EB), jnp.int32),
            pltpu.VMEM((2, 1, _EB), jnp.int32),
            pltpu.VMEM((_EB, channels), jnp.float32),
            pltpu.VMEM((_EB, channels), jnp.float32),
            pltpu.SemaphoreType.DMA,
            pltpu.SemaphoreType.DMA,
            pltpu.VMEM_SHARED((n_nodes + 8, channels), jnp.float32),
        ],
    )


def _transform_body(x_ref, w_ref, o_ref):
    o_ref[...] = lax.dot_general(
        x_ref[...], w_ref[...], (((1,), (1,)), ((), ())),
        preferred_element_type=jnp.float32)


def _gru_body(pp_ref, h_ref, wih_ref, whh_ref, bih_ref, bhh_ref, wlin_ref,
              h_out, t_out):
    ch = h_ref.shape[-1]
    prop = pp_ref[0] + pp_ref[1]
    h = h_ref[...]
    gi = lax.dot_general(prop, wih_ref[...], (((1,), (1,)), ((), ())),
                         preferred_element_type=jnp.float32) + bih_ref[...]
    gh = lax.dot_general(h, whh_ref[...], (((1,), (1,)), ((), ())),
                         preferred_element_type=jnp.float32) + bhh_ref[...]
    r = jax.nn.sigmoid(gi[:, :ch] + gh[:, :ch])
    z = jax.nn.sigmoid(gi[:, ch:2 * ch] + gh[:, ch:2 * ch])
    n = jnp.tanh(gi[:, 2 * ch:] + r * gh[:, 2 * ch:])
    hn = (1.0 - z) * n + z * h
    h_out[...] = hn
    t_out[...] = lax.dot_general(hn, wlin_ref[...], (((1,), (1,)), ((), ())),
                                 preferred_element_type=jnp.float32)


def _transform_call(x, w_lin, blk):
    n, ch = x.shape
    grid = n // blk
    return pl.pallas_call(
        _transform_body,
        grid=(grid,),
        in_specs=[
            pl.BlockSpec((blk, ch), lambda i: (i, 0)),
            pl.BlockSpec((ch, ch), lambda i: (0, 0)),
        ],
        out_specs=pl.BlockSpec((blk, ch), lambda i: (i, 0)),
        out_shape=jax.ShapeDtypeStruct((n, ch), jnp.float32),
    )(x, w_lin)


def _gru_call(pp, h, w_ih, w_hh, b_ih, b_hh, w_lin, blk):
    n, ch = h.shape
    grid = n // blk
    full = lambda i: (0, 0)
    return pl.pallas_call(
        _gru_body,
        grid=(grid,),
        in_specs=[
            pl.BlockSpec((_NC, blk, ch), lambda i: (0, i, 0)),
            pl.BlockSpec((blk, ch), lambda i: (i, 0)),
            pl.BlockSpec((3 * ch, ch), full),
            pl.BlockSpec((3 * ch, ch), full),
            pl.BlockSpec((1, 3 * ch), full),
            pl.BlockSpec((1, 3 * ch), full),
            pl.BlockSpec((ch, ch), full),
        ],
        out_specs=[
            pl.BlockSpec((blk, ch), lambda i: (i, 0)),
            pl.BlockSpec((blk, ch), lambda i: (i, 0)),
        ],
        out_shape=[
            jax.ShapeDtypeStruct((n, ch), jnp.float32),
            jax.ShapeDtypeStruct((n, ch), jnp.float32),
        ],
    )(pp, h, w_ih, w_hh, b_ih, b_hh, w_lin)


def kernel(x, edge_index, W_lin, W_ih, W_hh, b_ih, b_hh):
    n, ch = x.shape
    n_edges = edge_index.shape[1]
    grain = _NW * _EB
    n_edges_pad = -(-n_edges // grain) * grain
    nb = n_edges_pad // grain
    rpt = n // _NS
    pad = n_edges_pad - n_edges

    src = jnp.concatenate(
        [edge_index[0].astype(jnp.int32), jnp.zeros((pad,), jnp.int32)])
    dst = jnp.concatenate(
        [edge_index[1].astype(jnp.int32), jnp.full((pad,), n, jnp.int32)])
    src = src.reshape(_NW, nb, _EB)
    dst = dst.reshape(_NW, nb, 1, _EB)
    zeros = jnp.zeros((rpt, ch), jnp.float32)
    bih = b_ih.reshape(1, 3 * ch)
    bhh = b_hh.reshape(1, 3 * ch)

    propagate = _make_propagate(n, n_edges_pad, ch)
    blk = 2000

    t = _transform_call(x, W_lin, blk)
    state = x
    for _ in range(_STEPS):
        partials = propagate(src, dst, t, zeros)
        pp = partials.reshape(_NC, n, ch)
        state, t = _gru_call(pp, state, W_ih, W_hh, bih, bhh, W_lin, blk)
    return state


# speedup vs baseline: 2.3145x; 2.3145x over previous
import functools

import jax
import jax.numpy as jnp
from jax import lax
from jax.experimental import pallas as pl
from jax.experimental.pallas import tpu as pltpu
from jax.experimental.pallas import tpu_sc as plsc

_STEPS = 3
_NC, _NS = 2, 16
_NW = _NC * _NS
_EB = 80
_NSLOT = 3


def _propagate_body(n_nodes, nb, packed_hbm, table_hbm, zeros_hbm, out_hbm,
                    packed_v, src_ring, dst_ring, rows, gsems, ssems, acc_sh):
    c = lax.axis_index("c")
    s = lax.axis_index("s")
    w = s * _NC + c
    rpt = n_nodes // _NS

    pltpu.sync_copy(zeros_hbm, acc_sh.at[pl.ds(s * rpt, rpt)])

    pltpu.sync_copy(packed_hbm.at[w], packed_v)
    plsc.subcore_barrier()

    def unpack(j, p):
        for k in range(_EB // 16):
            v = packed_v[j, pl.ds(16 * k, 16)]
            src_ring[p, 0, pl.ds(16 * k, 16)] = lax.shift_right_logical(v, 16)
            dst_ring[p, 0, pl.ds(16 * k, 16)] = lax.bitwise_and(v, 0xFFFF)

    def start_gather(j, p):
        pltpu.async_copy(table_hbm.at[src_ring.at[p, 0]], rows[p], gsems[p])

    def wait_scatter(p):
        pltpu.make_async_copy(rows[p], acc_sh.at[dst_ring.at[p, 0]],
                              ssems[p]).wait()

    def finish(j, p):
        pltpu.make_async_copy(table_hbm.at[src_ring.at[p, 0]], rows[p],
                              gsems[p]).wait()
        pltpu.async_copy(rows[p], acc_sh.at[dst_ring.at[p, 0]], ssems[p],
                         add=True)

    for p in range(_NSLOT):
        unpack(p, p)
        start_gather(p, p)

    def body(i, _):
        j0 = _NSLOT * i
        for p in range(_NSLOT):
            j = j0 + p
            finish(j, p)

            @pl.when(j + _NSLOT < nb)
            def _():
                wait_scatter(p)
                unpack(j + _NSLOT, p)
                start_gather(j + _NSLOT, p)

        return 0

    lax.fori_loop(0, nb // _NSLOT, body, 0)
    for j in range(nb - nb % _NSLOT, nb):
        finish(j, j % _NSLOT)
    for p in range(_NSLOT):
        wait_scatter(p)
    plsc.subcore_barrier()

    pltpu.sync_copy(acc_sh.at[pl.ds(s * rpt, rpt)], out_hbm.at[c, s])


def _make_propagate(n_nodes, n_edges, channels):
    assert n_edges % (_NW * _EB) == 0
    nb = n_edges // (_NW * _EB)
    assert nb >= 2 * _NSLOT
    rpt = n_nodes // _NS
    mesh = plsc.VectorSubcoreMesh(core_axis_name="c", subcore_axis_name="s")

    def wrapped(packed_hbm, table_hbm, zeros_hbm, out_hbm, packed_v,
                src_ring, dst_ring, r0, r1, r2, g0, g1, g2, s0, s1, s2,
                acc_sh):
        _propagate_body(n_nodes, nb, packed_hbm, table_hbm, zeros_hbm,
                        out_hbm, packed_v, src_ring, dst_ring,
                        (r0, r1, r2), (g0, g1, g2), (s0, s1, s2), acc_sh)

    fn = pl.kernel(
        wrapped,
        out_type=jax.ShapeDtypeStruct((_NC, _NS, rpt, channels), jnp.float32),
        mesh=mesh,
        scratch_types=[
            pltpu.VMEM((nb, _EB), jnp.int32),
            pltpu.VMEM((_NSLOT, 1, _EB), jnp.int32),
            pltpu.VMEM((_NSLOT, 1, _EB), jnp.int32),
            pltpu.VMEM((_EB, channels), jnp.float32),
            pltpu.VMEM((_EB, channels), jnp.float32),
            pltpu.VMEM((_EB, channels), jnp.float32),
            pltpu.SemaphoreType.DMA,
            pltpu.SemaphoreType.DMA,
            pltpu.SemaphoreType.DMA,
            pltpu.SemaphoreType.DMA,
            pltpu.SemaphoreType.DMA,
            pltpu.SemaphoreType.DMA,
            pltpu.VMEM_SHARED((n_nodes, channels), jnp.float32),
        ],
    )
    return fn, nb


def _transform_body(x_ref, w_ref, o_ref):
    o_ref[...] = lax.dot_general(
        x_ref[...], w_ref[...], (((1,), (1,)), ((), ())),
        preferred_element_type=jnp.float32)


def _gru_body(pp_ref, h_ref, wih_ref, whh_ref, bih_ref, bhh_ref, wlin_ref,
              h_out, t_out):
    ch = h_ref.shape[-1]
    prop = pp_ref[0] + pp_ref[1]
    h = h_ref[...]
    gi = lax.dot_general(prop, wih_ref[...], (((1,), (1,)), ((), ())),
                         preferred_element_type=jnp.float32) + bih_ref[...]
    gh = lax.dot_general(h, whh_ref[...], (((1,), (1,)), ((), ())),
                         preferred_element_type=jnp.float32) + bhh_ref[...]
    r = jax.nn.sigmoid(gi[:, :ch] + gh[:, :ch])
    z = jax.nn.sigmoid(gi[:, ch:2 * ch] + gh[:, ch:2 * ch])
    n = jnp.tanh(gi[:, 2 * ch:] + r * gh[:, 2 * ch:])
    hn = (1.0 - z) * n + z * h
    h_out[...] = hn
    t_out[...] = lax.dot_general(hn, wlin_ref[...], (((1,), (1,)), ((), ())),
                                 preferred_element_type=jnp.float32)


def _transform_call(x, w_lin, blk):
    n, ch = x.shape
    grid = n // blk
    return pl.pallas_call(
        _transform_body,
        grid=(grid,),
        in_specs=[
            pl.BlockSpec((blk, ch), lambda i: (i, 0)),
            pl.BlockSpec((ch, ch), lambda i: (0, 0)),
        ],
        out_specs=pl.BlockSpec((blk, ch), lambda i: (i, 0)),
        out_shape=jax.ShapeDtypeStruct((n, ch), jnp.float32),
    )(x, w_lin)


def _gru_call(pp, h, w_ih, w_hh, b_ih, b_hh, w_lin, blk):
    n, ch = h.shape
    grid = n // blk
    full = lambda i: (0, 0)
    return pl.pallas_call(
        _gru_body,
        grid=(grid,),
        in_specs=[
            pl.BlockSpec((_NC, blk, ch), lambda i: (0, i, 0)),
            pl.BlockSpec((blk, ch), lambda i: (i, 0)),
            pl.BlockSpec((3 * ch, ch), full),
            pl.BlockSpec((3 * ch, ch), full),
            pl.BlockSpec((1, 3 * ch), full),
            pl.BlockSpec((1, 3 * ch), full),
            pl.BlockSpec((ch, ch), full),
        ],
        out_specs=[
            pl.BlockSpec((blk, ch), lambda i: (i, 0)),
            pl.BlockSpec((blk, ch), lambda i: (i, 0)),
        ],
        out_shape=[
            jax.ShapeDtypeStruct((n, ch), jnp.float32),
            jax.ShapeDtypeStruct((n, ch), jnp.float32),
        ],
    )(pp, h, w_ih, w_hh, b_ih, b_hh, w_lin)


def kernel(x, edge_index, W_lin, W_ih, W_hh, b_ih, b_hh):
    n, ch = x.shape
    n_edges = edge_index.shape[1]
    rpt = n // _NS

    propagate, nb = _make_propagate(n, n_edges, ch)

    src32 = edge_index[0].astype(jnp.int32)
    dst32 = edge_index[1].astype(jnp.int32)
    packed = (src32 * 65536 + dst32).reshape(_NW, nb, _EB)
    zeros = jnp.zeros((rpt, ch), jnp.float32)
    bih = b_ih.reshape(1, 3 * ch)
    bhh = b_hh.reshape(1, 3 * ch)

    blk = 2000
    t = _transform_call(x, W_lin, blk)
    state = x
    for _ in range(_STEPS):
        partials = propagate(packed, t, zeros)
        pp = partials.reshape(_NC, n, ch)
        state, t = _gru_call(pp, state, W_ih, W_hh, bih, bhh, W_lin, blk)
    return state


# speedup vs baseline: 2.3199x; 1.0023x over previous
import functools

import jax
import jax.numpy as jnp
from jax import lax
from jax.experimental import pallas as pl
from jax.experimental.pallas import tpu as pltpu
from jax.experimental.pallas import tpu_sc as plsc

_STEPS = 3
_NC, _NS = 2, 16
_NW = _NC * _NS
_EB = 80
_NSLOT = 3


def _propagate_body(n_nodes, nb, packed_hbm, table_hbm, zeros_hbm, out_hbm,
                    packed_v, src_ring, dst_ring, rows, gsems, ssems, acc_sh):
    c = lax.axis_index("c")
    s = lax.axis_index("s")
    w = s * _NC + c
    rpt = n_nodes // _NS

    pltpu.sync_copy(zeros_hbm, acc_sh.at[pl.ds(s * rpt, rpt)])

    pltpu.sync_copy(packed_hbm.at[w], packed_v)
    plsc.subcore_barrier()

    def unpack(j, p):
        for k in range(_EB // 16):
            v = packed_v[j, pl.ds(16 * k, 16)]
            src_ring[p, 0, pl.ds(16 * k, 16)] = lax.shift_right_logical(v, 16)
            dst_ring[p, 0, pl.ds(16 * k, 16)] = lax.bitwise_and(v, 0xFFFF)

    def start_gather(j, p):
        pltpu.async_copy(table_hbm.at[src_ring.at[p, 0]], rows[p], gsems[p])

    def wait_scatter(p):
        pltpu.make_async_copy(rows[p], acc_sh.at[dst_ring.at[p, 0]],
                              ssems[p]).wait()

    def finish(j, p):
        pltpu.make_async_copy(table_hbm.at[src_ring.at[p, 0]], rows[p],
                              gsems[p]).wait()
        pltpu.async_copy(rows[p], acc_sh.at[dst_ring.at[p, 0]], ssems[p],
                         add=True)

    for p in range(_NSLOT):
        unpack(p, p)
        start_gather(p, p)

    def body(i, _):
        j0 = _NSLOT * i
        for p in range(_NSLOT):
            j = j0 + p
            finish(j, p)

            @pl.when(j + _NSLOT < nb)
            def _():
                wait_scatter(p)
                unpack(j + _NSLOT, p)
                start_gather(j + _NSLOT, p)

        return 0

    lax.fori_loop(0, nb // _NSLOT, body, 0)
    for j in range(nb - nb % _NSLOT, nb):
        finish(j, j % _NSLOT)
    for p in range(_NSLOT):
        wait_scatter(p)
    plsc.subcore_barrier()

    pltpu.sync_copy(acc_sh.at[pl.ds(s * rpt, rpt)], out_hbm.at[c, s])


def _make_propagate(n_nodes, n_edges, channels):
    assert n_edges % (_NW * _EB) == 0
    nb = n_edges // (_NW * _EB)
    assert nb >= 2 * _NSLOT
    rpt = n_nodes // _NS
    mesh = plsc.VectorSubcoreMesh(core_axis_name="c", subcore_axis_name="s")

    def wrapped(packed_hbm, table_hbm, zeros_hbm, out_hbm, packed_v,
                src_ring, dst_ring, r0, r1, r2, g0, g1, g2, s0, s1, s2,
                acc_sh):
        _propagate_body(n_nodes, nb, packed_hbm, table_hbm, zeros_hbm,
                        out_hbm, packed_v, src_ring, dst_ring,
                        (r0, r1, r2), (g0, g1, g2), (s0, s1, s2), acc_sh)

    fn = pl.kernel(
        wrapped,
        out_type=jax.ShapeDtypeStruct((_NC, _NS, rpt, channels), jnp.float32),
        mesh=mesh,
        scratch_types=[
            pltpu.VMEM((nb, _EB), jnp.int32),
            pltpu.VMEM((_NSLOT, 1, _EB), jnp.int32),
            pltpu.VMEM((_NSLOT, 1, _EB), jnp.int32),
            pltpu.VMEM((_EB, channels), jnp.float32),
            pltpu.VMEM((_EB, channels), jnp.float32),
            pltpu.VMEM((_EB, channels), jnp.float32),
            pltpu.SemaphoreType.DMA,
            pltpu.SemaphoreType.DMA,
            pltpu.SemaphoreType.DMA,
            pltpu.SemaphoreType.DMA,
            pltpu.SemaphoreType.DMA,
            pltpu.SemaphoreType.DMA,
            pltpu.VMEM_SHARED((n_nodes, channels), jnp.float32),
        ],
    )
    return fn, nb


def _transform_body(x_ref, w_ref, o_ref):
    o_ref[...] = lax.dot_general(
        x_ref[...], w_ref[...], (((1,), (1,)), ((), ())),
        preferred_element_type=jnp.float32)


def _gru_body(with_transform, pp_ref, h_ref, wih_ref, whh_ref, bih_ref,
              bhh_ref, wlin_ref, h_out, *maybe_t_out):
    ch = h_ref.shape[-1]
    prop = pp_ref[0] + pp_ref[1]
    h = h_ref[...]
    gi = lax.dot_general(prop, wih_ref[...], (((1,), (1,)), ((), ())),
                         preferred_element_type=jnp.float32) + bih_ref[...]
    gh = lax.dot_general(h, whh_ref[...], (((1,), (1,)), ((), ())),
                         preferred_element_type=jnp.float32) + bhh_ref[...]
    r = jax.nn.sigmoid(gi[:, :ch] + gh[:, :ch])
    z = jax.nn.sigmoid(gi[:, ch:2 * ch] + gh[:, ch:2 * ch])
    n = jnp.tanh(gi[:, 2 * ch:] + r * gh[:, 2 * ch:])
    hn = (1.0 - z) * n + z * h
    h_out[...] = hn
    if with_transform:
        maybe_t_out[0][...] = lax.dot_general(
            hn, wlin_ref[...], (((1,), (1,)), ((), ())),
            preferred_element_type=jnp.float32)


def _transform_call(x, w_lin, blk):
    n, ch = x.shape
    grid = n // blk
    return pl.pallas_call(
        _transform_body,
        grid=(grid,),
        in_specs=[
            pl.BlockSpec((blk, ch), lambda i: (i, 0)),
            pl.BlockSpec((ch, ch), lambda i: (0, 0)),
        ],
        out_specs=pl.BlockSpec((blk, ch), lambda i: (i, 0)),
        out_shape=jax.ShapeDtypeStruct((n, ch), jnp.float32),
    )(x, w_lin)


def _gru_call(pp, h, w_ih, w_hh, b_ih, b_hh, w_lin, blk, with_transform):
    n, ch = h.shape
    grid = n // blk
    full = lambda i: (0, 0)
    n_out = 2 if with_transform else 1
    out = pl.pallas_call(
        functools.partial(_gru_body, with_transform),
        grid=(grid,),
        in_specs=[
            pl.BlockSpec((_NC, blk, ch), lambda i: (0, i, 0)),
            pl.BlockSpec((blk, ch), lambda i: (i, 0)),
            pl.BlockSpec((3 * ch, ch), full),
            pl.BlockSpec((3 * ch, ch), full),
            pl.BlockSpec((1, 3 * ch), full),
            pl.BlockSpec((1, 3 * ch), full),
            pl.BlockSpec((ch, ch), full),
        ],
        out_specs=[pl.BlockSpec((blk, ch), lambda i: (i, 0))] * n_out,
        out_shape=[jax.ShapeDtypeStruct((n, ch), jnp.float32)] * n_out,
    )(pp, h, w_ih, w_hh, b_ih, b_hh, w_lin)
    return out if with_transform else (out[0], None)


def kernel(x, edge_index, W_lin, W_ih, W_hh, b_ih, b_hh):
    n, ch = x.shape
    n_edges = edge_index.shape[1]
    rpt = n // _NS

    propagate, nb = _make_propagate(n, n_edges, ch)

    src32 = edge_index[0].astype(jnp.int32)
    dst32 = edge_index[1].astype(jnp.int32)
    packed = (src32 * 65536 + dst32).reshape(_NW, nb, _EB)
    zeros = jnp.zeros((rpt, ch), jnp.float32)
    bih = b_ih.reshape(1, 3 * ch)
    bhh = b_hh.reshape(1, 3 * ch)

    blk = 2000
    t = _transform_call(x, W_lin, blk)
    state = x
    for step in range(_STEPS):
        partials = propagate(packed, t, zeros)
        pp = partials.reshape(_NC, n, ch)
        state, t = _gru_call(pp, state, W_ih, W_hh, bih, bhh, W_lin, blk,
                             with_transform=step < _STEPS - 1)
    return state


# speedup vs baseline: 2.3782x; 1.0251x over previous
import functools

import jax
import jax.numpy as jnp
from jax import lax
from jax.experimental import pallas as pl
from jax.experimental.pallas import tpu as pltpu
from jax.experimental.pallas import tpu_sc as plsc

_STEPS = 3
_NC, _NS = 2, 16
_NW = _NC * _NS
_EB = 80
_NSLOT = 4
_BK = 25


def _propagate_body(n_nodes, nb, packed_hbm, table_hbm, zeros_hbm, out_hbm,
                    packed_v, src_ring, dst_ring, rows, gsems, ssems, bsem,
                    acc_sh):
    c = lax.axis_index("c")
    s = lax.axis_index("s")
    w = s * _NC + c
    rpt = n_nodes // _NS
    n_blk = nb // _BK

    pltpu.sync_copy(zeros_hbm, acc_sh.at[pl.ds(s * rpt, rpt)])

    def fetch_block(m):
        pltpu.async_copy(packed_hbm.at[w, m], packed_v.at[m % 2], bsem)

    def wait_block(m):
        pltpu.make_async_copy(packed_hbm.at[w, m], packed_v.at[m % 2],
                              bsem).wait()

    fetch_block(0)
    plsc.subcore_barrier()

    def unpack(u, p):
        u = jnp.asarray(u, jnp.int32)
        m = u // _BK
        r = u % _BK

        @pl.when(r == 0)
        def _():
            wait_block(m)

            @pl.when(m + 1 < n_blk)
            def _():
                fetch_block(m + 1)

        for k in range(_EB // 16):
            v = packed_v[m % 2, r, pl.ds(16 * k, 16)]
            src_ring[p, 0, pl.ds(16 * k, 16)] = lax.shift_right_logical(v, 16)
            dst_ring[p, 0, pl.ds(16 * k, 16)] = lax.bitwise_and(v, 0xFFFF)

    def start_gather(j, p):
        pltpu.async_copy(table_hbm.at[src_ring.at[p, 0]], rows[p], gsems[p])

    def wait_scatter(p):
        pltpu.make_async_copy(rows[p], acc_sh.at[dst_ring.at[p, 0]],
                              ssems[p]).wait()

    def finish(j, p):
        pltpu.make_async_copy(table_hbm.at[src_ring.at[p, 0]], rows[p],
                              gsems[p]).wait()
        pltpu.async_copy(rows[p], acc_sh.at[dst_ring.at[p, 0]], ssems[p],
                         add=True)

    for p in range(_NSLOT):
        unpack(p, p)
        start_gather(p, p)

    def body(i, _):
        j0 = _NSLOT * i
        for p in range(_NSLOT):
            j = j0 + p
            finish(j, p)

            @pl.when(j + _NSLOT < nb)
            def _():
                wait_scatter(p)
                unpack(j + _NSLOT, p)
                start_gather(j + _NSLOT, p)

        return 0

    lax.fori_loop(0, nb // _NSLOT, body, 0)
    for j in range(nb - nb % _NSLOT, nb):
        finish(j, j % _NSLOT)
    for p in range(_NSLOT):
        wait_scatter(p)
    plsc.subcore_barrier()

    pltpu.sync_copy(acc_sh.at[pl.ds(s * rpt, rpt)], out_hbm.at[c, s])


def _make_propagate(n_nodes, n_edges, channels):
    assert n_edges % (_NW * _EB) == 0
    nb = n_edges // (_NW * _EB)
    assert nb >= 2 * _NSLOT and nb % _BK == 0
    rpt = n_nodes // _NS
    mesh = plsc.VectorSubcoreMesh(core_axis_name="c", subcore_axis_name="s")

    def wrapped(packed_hbm, table_hbm, zeros_hbm, out_hbm, packed_v,
                src_ring, dst_ring, r0, r1, r2, r3, g0, g1, g2, g3,
                s0, s1, s2, s3, bsem, acc_sh):
        _propagate_body(n_nodes, nb, packed_hbm, table_hbm, zeros_hbm,
                        out_hbm, packed_v, src_ring, dst_ring,
                        (r0, r1, r2, r3), (g0, g1, g2, g3),
                        (s0, s1, s2, s3), bsem, acc_sh)

    fn = pl.kernel(
        wrapped,
        out_type=jax.ShapeDtypeStruct((_NC, _NS, rpt, channels), jnp.float32),
        mesh=mesh,
        scratch_types=[
            pltpu.VMEM((2, _BK, _EB), jnp.int32),
            pltpu.VMEM((_NSLOT, 1, _EB), jnp.int32),
            pltpu.VMEM((_NSLOT, 1, _EB), jnp.int32),
            pltpu.VMEM((_EB, channels), jnp.float32),
            pltpu.VMEM((_EB, channels), jnp.float32),
            pltpu.VMEM((_EB, channels), jnp.float32),
            pltpu.VMEM((_EB, channels), jnp.float32),
            pltpu.SemaphoreType.DMA,
            pltpu.SemaphoreType.DMA,
            pltpu.SemaphoreType.DMA,
            pltpu.SemaphoreType.DMA,
            pltpu.SemaphoreType.DMA,
            pltpu.SemaphoreType.DMA,
            pltpu.SemaphoreType.DMA,
            pltpu.SemaphoreType.DMA,
            pltpu.SemaphoreType.DMA,
            pltpu.VMEM_SHARED((n_nodes, channels), jnp.float32),
        ],
    )
    return fn, nb


def _transform_body(x_ref, w_ref, o_ref):
    o_ref[...] = lax.dot_general(
        x_ref[...], w_ref[...], (((1,), (1,)), ((), ())),
        preferred_element_type=jnp.float32)


def _gru_body(with_transform, pp_ref, h_ref, wih_ref, whh_ref, bih_ref,
              bhh_ref, wlin_ref, h_out, *maybe_t_out):
    ch = h_ref.shape[-1]
    prop = pp_ref[0] + pp_ref[1]
    h = h_ref[...]
    gi = lax.dot_general(prop, wih_ref[...], (((1,), (1,)), ((), ())),
                         preferred_element_type=jnp.float32) + bih_ref[...]
    gh = lax.dot_general(h, whh_ref[...], (((1,), (1,)), ((), ())),
                         preferred_element_type=jnp.float32) + bhh_ref[...]
    r = jax.nn.sigmoid(gi[:, :ch] + gh[:, :ch])
    z = jax.nn.sigmoid(gi[:, ch:2 * ch] + gh[:, ch:2 * ch])
    n = jnp.tanh(gi[:, 2 * ch:] + r * gh[:, 2 * ch:])
    hn = (1.0 - z) * n + z * h
    h_out[...] = hn
    if with_transform:
        maybe_t_out[0][...] = lax.dot_general(
            hn, wlin_ref[...], (((1,), (1,)), ((), ())),
            preferred_element_type=jnp.float32)


def _transform_call(x, w_lin, blk):
    n, ch = x.shape
    grid = n // blk
    return pl.pallas_call(
        _transform_body,
        grid=(grid,),
        in_specs=[
            pl.BlockSpec((blk, ch), lambda i: (i, 0)),
            pl.BlockSpec((ch, ch), lambda i: (0, 0)),
        ],
        out_specs=pl.BlockSpec((blk, ch), lambda i: (i, 0)),
        out_shape=jax.ShapeDtypeStruct((n, ch), jnp.float32),
    )(x, w_lin)


def _gru_call(pp, h, w_ih, w_hh, b_ih, b_hh, w_lin, blk, with_transform):
    n, ch = h.shape
    grid = n // blk
    full = lambda i: (0, 0)
    n_out = 2 if with_transform else 1
    out = pl.pallas_call(
        functools.partial(_gru_body, with_transform),
        grid=(grid,),
        in_specs=[
            pl.BlockSpec((_NC, blk, ch), lambda i: (0, i, 0)),
            pl.BlockSpec((blk, ch), lambda i: (i, 0)),
            pl.BlockSpec((3 * ch, ch), full),
            pl.BlockSpec((3 * ch, ch), full),
            pl.BlockSpec((1, 3 * ch), full),
            pl.BlockSpec((1, 3 * ch), full),
            pl.BlockSpec((ch, ch), full),
        ],
        out_specs=[pl.BlockSpec((blk, ch), lambda i: (i, 0))] * n_out,
        out_shape=[jax.ShapeDtypeStruct((n, ch), jnp.float32)] * n_out,
    )(pp, h, w_ih, w_hh, b_ih, b_hh, w_lin)
    return out if with_transform else (out[0], None)


def kernel(x, edge_index, W_lin, W_ih, W_hh, b_ih, b_hh):
    n, ch = x.shape
    n_edges = edge_index.shape[1]
    rpt = n // _NS

    propagate, nb = _make_propagate(n, n_edges, ch)

    src32 = edge_index[0].astype(jnp.int32)
    dst32 = edge_index[1].astype(jnp.int32)
    packed = (src32 * 65536 + dst32).reshape(_NW, nb // _BK, _BK, _EB)
    zeros = jnp.zeros((rpt, ch), jnp.float32)
    bih = b_ih.reshape(1, 3 * ch)
    bhh = b_hh.reshape(1, 3 * ch)

    blk = 2000
    t = _transform_call(x, W_lin, blk)
    state = x
    for step in range(_STEPS):
        partials = propagate(packed, t, zeros)
        pp = partials.reshape(_NC, n, ch)
        state, t = _gru_call(pp, state, W_ih, W_hh, bih, bhh, W_lin, blk,
                             with_transform=step < _STEPS - 1)
    return state


# speedup vs baseline: 2.3910x; 1.0053x over previous
import functools

import jax
import jax.numpy as jnp
from jax import lax
from jax.experimental import pallas as pl
from jax.experimental.pallas import tpu as pltpu
from jax.experimental.pallas import tpu_sc as plsc

_STEPS = 3
_NC, _NS = 2, 16
_NW = _NC * _NS
_EB = 80
_NSLOT = 4
_BK = 25


def _propagate_body(n_nodes, nb, packed_hbm, table_hbm, zeros_hbm, out_hbm,
                    packed_v, src_ring, dst_ring, rows, gsems, ssems, bsem,
                    acc_sh):
    c = lax.axis_index("c")
    s = lax.axis_index("s")
    w = s * _NC + c
    rpt = n_nodes // _NS
    n_blk = nb // _BK

    def fetch_block(m):
        pltpu.async_copy(packed_hbm.at[w, m], packed_v.at[m % 2], bsem)

    def wait_block(m):
        pltpu.make_async_copy(packed_hbm.at[w, m], packed_v.at[m % 2],
                              bsem).wait()

    fetch_block(0)
    pltpu.sync_copy(zeros_hbm, acc_sh.at[pl.ds(s * rpt, rpt)])
    plsc.subcore_barrier()

    def unpack(u, p):
        u = jnp.asarray(u, jnp.int32)
        m = u // _BK
        r = u % _BK

        @pl.when(r == 0)
        def _():
            wait_block(m)

            @pl.when(m + 1 < n_blk)
            def _():
                fetch_block(m + 1)

        for k in range(_EB // 16):
            v = packed_v[m % 2, r, pl.ds(16 * k, 16)]
            src_ring[p, 0, pl.ds(16 * k, 16)] = lax.shift_right_logical(v, 16)
            dst_ring[p, 0, pl.ds(16 * k, 16)] = lax.bitwise_and(v, 0xFFFF)

    def start_gather(j, p):
        pltpu.async_copy(table_hbm.at[src_ring.at[p, 0]], rows[p], gsems[p])

    def wait_scatter(p):
        pltpu.make_async_copy(rows[p], acc_sh.at[dst_ring.at[p, 0]],
                              ssems[p]).wait()

    def finish(j, p):
        pltpu.make_async_copy(table_hbm.at[src_ring.at[p, 0]], rows[p],
                              gsems[p]).wait()
        pltpu.async_copy(rows[p], acc_sh.at[dst_ring.at[p, 0]], ssems[p],
                         add=True)

    for p in range(_NSLOT):
        unpack(p, p)
        start_gather(p, p)

    def body(i, _):
        j0 = _NSLOT * i
        for p in range(_NSLOT):
            j = j0 + p
            finish(j, p)

            @pl.when(j + _NSLOT < nb)
            def _():
                wait_scatter(p)
                unpack(j + _NSLOT, p)
                start_gather(j + _NSLOT, p)

        return 0

    lax.fori_loop(0, nb // _NSLOT, body, 0)
    for j in range(nb - nb % _NSLOT, nb):
        finish(j, j % _NSLOT)
    for p in range(_NSLOT):
        wait_scatter(p)
    plsc.subcore_barrier()

    pltpu.sync_copy(acc_sh.at[pl.ds(s * rpt, rpt)], out_hbm.at[c, s])


def _make_propagate(n_nodes, n_edges, channels):
    assert n_edges % (_NW * _EB) == 0
    nb = n_edges // (_NW * _EB)
    assert nb >= 2 * _NSLOT and nb % _BK == 0
    rpt = n_nodes // _NS
    mesh = plsc.VectorSubcoreMesh(core_axis_name="c", subcore_axis_name="s")

    def wrapped(packed_hbm, table_hbm, zeros_hbm, out_hbm, packed_v,
                src_ring, dst_ring, r0, r1, r2, r3, g0, g1, g2, g3,
                s0, s1, s2, s3, bsem, acc_sh):
        _propagate_body(n_nodes, nb, packed_hbm, table_hbm, zeros_hbm,
                        out_hbm, packed_v, src_ring, dst_ring,
                        (r0, r1, r2, r3), (g0, g1, g2, g3),
                        (s0, s1, s2, s3), bsem, acc_sh)

    fn = pl.kernel(
        wrapped,
        out_type=jax.ShapeDtypeStruct((_NC, _NS, rpt, channels), jnp.float32),
        mesh=mesh,
        scratch_types=[
            pltpu.VMEM((2, _BK, _EB), jnp.int32),
            pltpu.VMEM((_NSLOT, 1, _EB), jnp.int32),
            pltpu.VMEM((_NSLOT, 1, _EB), jnp.int32),
            pltpu.VMEM((_EB, channels), jnp.float32),
            pltpu.VMEM((_EB, channels), jnp.float32),
            pltpu.VMEM((_EB, channels), jnp.float32),
            pltpu.VMEM((_EB, channels), jnp.float32),
            pltpu.SemaphoreType.DMA,
            pltpu.SemaphoreType.DMA,
            pltpu.SemaphoreType.DMA,
            pltpu.SemaphoreType.DMA,
            pltpu.SemaphoreType.DMA,
            pltpu.SemaphoreType.DMA,
            pltpu.SemaphoreType.DMA,
            pltpu.SemaphoreType.DMA,
            pltpu.SemaphoreType.DMA,
            pltpu.VMEM_SHARED((n_nodes, channels), jnp.float32),
        ],
    )
    return fn, nb


def _transform_body(x_ref, w_ref, o_ref):
    o_ref[...] = lax.dot_general(
        x_ref[...], w_ref[...], (((1,), (1,)), ((), ())),
        preferred_element_type=jnp.float32)


def _gru_body(with_transform, pp_ref, h_ref, wih_ref, whh_ref, bih_ref,
              bhh_ref, wlin_ref, h_out, *maybe_t_out):
    ch = h_ref.shape[-1]
    prop = pp_ref[0] + pp_ref[1]
    h = h_ref[...]
    gi = lax.dot_general(prop, wih_ref[...], (((1,), (1,)), ((), ())),
                         preferred_element_type=jnp.float32) + bih_ref[...]
    gh = lax.dot_general(h, whh_ref[...], (((1,), (1,)), ((), ())),
                         preferred_element_type=jnp.float32) + bhh_ref[...]
    r = jax.nn.sigmoid(gi[:, :ch] + gh[:, :ch])
    z = jax.nn.sigmoid(gi[:, ch:2 * ch] + gh[:, ch:2 * ch])
    n = jnp.tanh(gi[:, 2 * ch:] + r * gh[:, 2 * ch:])
    hn = (1.0 - z) * n + z * h
    h_out[...] = hn
    if with_transform:
        maybe_t_out[0][...] = lax.dot_general(
            hn, wlin_ref[...], (((1,), (1,)), ((), ())),
            preferred_element_type=jnp.float32)


def _transform_call(x, w_lin, blk):
    n, ch = x.shape
    grid = n // blk
    return pl.pallas_call(
        _transform_body,
        grid=(grid,),
        in_specs=[
            pl.BlockSpec((blk, ch), lambda i: (i, 0)),
            pl.BlockSpec((ch, ch), lambda i: (0, 0)),
        ],
        out_specs=pl.BlockSpec((blk, ch), lambda i: (i, 0)),
        out_shape=jax.ShapeDtypeStruct((n, ch), jnp.float32),
    )(x, w_lin)


def _gru_call(pp, h, w_ih, w_hh, b_ih, b_hh, w_lin, blk, with_transform):
    n, ch = h.shape
    grid = n // blk
    full = lambda i: (0, 0)
    n_out = 2 if with_transform else 1
    out = pl.pallas_call(
        functools.partial(_gru_body, with_transform),
        grid=(grid,),
        in_specs=[
            pl.BlockSpec((_NC, blk, ch), lambda i: (0, i, 0)),
            pl.BlockSpec((blk, ch), lambda i: (i, 0)),
            pl.BlockSpec((3 * ch, ch), full),
            pl.BlockSpec((3 * ch, ch), full),
            pl.BlockSpec((1, 3 * ch), full),
            pl.BlockSpec((1, 3 * ch), full),
            pl.BlockSpec((ch, ch), full),
        ],
        out_specs=[pl.BlockSpec((blk, ch), lambda i: (i, 0))] * n_out,
        out_shape=[jax.ShapeDtypeStruct((n, ch), jnp.float32)] * n_out,
    )(pp, h, w_ih, w_hh, b_ih, b_hh, w_lin)
    return out if with_transform else (out[0], None)


def kernel(x, edge_index, W_lin, W_ih, W_hh, b_ih, b_hh):
    n, ch = x.shape
    n_edges = edge_index.shape[1]
    rpt = n // _NS

    propagate, nb = _make_propagate(n, n_edges, ch)

    src32 = edge_index[0].astype(jnp.int32)
    dst32 = edge_index[1].astype(jnp.int32)
    packed = (src32 * 65536 + dst32).reshape(_NW, nb // _BK, _BK, _EB)
    zeros = jnp.zeros((rpt, ch), jnp.float32)
    bih = b_ih.reshape(1, 3 * ch)
    bhh = b_hh.reshape(1, 3 * ch)

    blk = 2000
    t = _transform_call(x, W_lin, blk)
    state = x
    for step in range(_STEPS):
        partials = propagate(packed, t, zeros)
        pp = partials.reshape(_NC, n, ch)
        state, t = _gru_call(pp, state, W_ih, W_hh, bih, bhh, W_lin, blk,
                             with_transform=step < _STEPS - 1)
    return state
